# Initial kernel scaffold; baseline (speedup 1.0000x reference)
import jax
import jax.numpy as jnp
from jax.experimental import pallas as pl


def kernel(x, edge_index, edge_attr, batch, W1, b1, W2, b2, W3, b3, root, bias1, Wg, att_src, att_dst, bias_g, fc1_w, fc1_b, fc2_w, fc2_b):
    raise NotImplementedError("write your pallas kernel here")


# speedup vs baseline: 13.4878x; 13.4878x over previous
import functools

import jax
import jax.numpy as jnp
from jax import lax
from jax.experimental import pallas as pl
from jax.experimental.pallas import tpu as pltpu
from jax.experimental.pallas import tpu_sc as plsc

N = 10000
E = 160000
F = 16
H = 16
G = 64
O = 64

NC = 2
NS = 16
L = 16
NW = NC * NS

BPW = E // NW
BPW_UP = 5008
N_PAD = 10240
STRIPE = N_PAD // NS

BE = 2000
GE = E // BE
BN = 1280
GN = N_PAD // BN

_SC_PARAMS = pltpu.CompilerParams(use_tc_tiling_on_sc=False)


def _mesh():
    return plsc.VectorSubcoreMesh(core_axis_name="c", subcore_axis_name="s",
                                  num_cores=NC, num_subcores=NS)


@functools.cache
def _build_sc_gather_rows():
  kern = functools.partial(
    pl.kernel,
    out_type=jax.ShapeDtypeStruct((E, F), jnp.float32),
    mesh=_mesh(),
    compiler_params=_SC_PARAMS,
    scratch_types=[
        pltpu.VMEM((BPW,), jnp.int32),
        pltpu.VMEM((BPW, F), jnp.float32),
        pltpu.SemaphoreType.DMA,
    ],
  )
  @kern
  def _sc_gather_rows(x_hbm, src_hbm, out_hbm, idx_v, rows_v, sem):
    wid = lax.axis_index("s") * NC + lax.axis_index("c")
    base = wid * BPW
    pltpu.sync_copy(src_hbm.at[pl.ds(base, BPW)], idx_v)
    pltpu.async_copy(x_hbm.at[idx_v], rows_v, sem).wait()
    pltpu.sync_copy(rows_v, out_hbm.at[pl.ds(base, BPW)])

  return _sc_gather_rows


@functools.cache
def _build_sc_scatter_add_rows():
  kern = functools.partial(
    pl.kernel,
    out_type=jax.ShapeDtypeStruct((NC, N_PAD, F), jnp.float32),
    mesh=_mesh(),
    compiler_params=_SC_PARAMS,
    scratch_types=[
        pltpu.VMEM((BPW,), jnp.int32),
        pltpu.VMEM((BPW, F), jnp.float32),
        pltpu.VMEM((STRIPE, F), jnp.float32),
        pltpu.VMEM_SHARED((N_PAD, F), jnp.float32),
        pltpu.SemaphoreType.DMA,
        pltpu.SemaphoreType.DMA,
    ],
  )
  @kern
  def _sc_scatter_add_rows(msg_hbm, dst_hbm, out_hbm, idx_v, rows_v, zb_v,
                           acc_sh, sem, sem2):
    c = lax.axis_index("c")
    s = lax.axis_index("s")
    wid = s * NC + c
    base = wid * BPW

    cp_idx = pltpu.async_copy(dst_hbm.at[pl.ds(base, BPW)], idx_v, sem)
    cp_rows = pltpu.async_copy(msg_hbm.at[pl.ds(base, BPW)], rows_v, sem2)

    def zrow(i, carry):
        zb_v[i, :] = jnp.zeros((L,), jnp.float32)
        return carry

    lax.fori_loop(0, STRIPE, zrow, 0)
    pltpu.sync_copy(zb_v, acc_sh.at[pl.ds(s * STRIPE, STRIPE)])
    cp_idx.wait()
    cp_rows.wait()
    plsc.subcore_barrier()
    pltpu.sync_copy(rows_v, acc_sh.at[idx_v], add=True)
    plsc.subcore_barrier()
    pltpu.sync_copy(
        acc_sh.at[pl.ds(s * STRIPE, STRIPE)],
        out_hbm.at[c, pl.ds(s * STRIPE, STRIPE)],
    )

  return _sc_scatter_add_rows


@functools.cache
def _build_sc_gat_edges():
  kern = functools.partial(
    pl.kernel,
    out_type=[
        jax.ShapeDtypeStruct((NC, N_PAD, F), jnp.float32),
        jax.ShapeDtypeStruct((NC, N_PAD), jnp.float32),
    ],
    mesh=_mesh(),
    compiler_params=_SC_PARAMS,
    scratch_types=[
        pltpu.VMEM((BPW,), jnp.int32),
        pltpu.VMEM((BPW,), jnp.int32),
        pltpu.VMEM((BPW_UP,), jnp.float32),
        pltpu.VMEM((BPW_UP,), jnp.float32),
        pltpu.VMEM((BPW_UP, F), jnp.float32),
        pltpu.VMEM((STRIPE, F), jnp.float32),
        pltpu.VMEM((STRIPE,), jnp.float32),
        pltpu.VMEM_SHARED((N_PAD, F), jnp.float32),
        pltpu.VMEM_SHARED((N_PAD,), jnp.float32),
        pltpu.SemaphoreType.DMA,
        pltpu.SemaphoreType.DMA,
        pltpu.SemaphoreType.DMA,
    ],
  )
  @kern
  def _sc_gat_edges(asrc_hbm, adst_hbm, xp_hbm, src_hbm, dst_hbm,
                    num_hbm, den_hbm,
                    si_v, di_v, av, bv, rows_v, zb_v, zd_v, num_sh, den_sh,
                    sem_a, sem_b, sem_r):
    c = lax.axis_index("c")
    s = lax.axis_index("s")
    wid = s * NC + c
    base = wid * BPW

    cp_si = pltpu.async_copy(src_hbm.at[pl.ds(base, BPW)], si_v, sem_a)
    cp_di = pltpu.async_copy(dst_hbm.at[pl.ds(base, BPW)], di_v, sem_b)

    def zrow(i, carry):
        zb_v[i, :] = jnp.zeros((L,), jnp.float32)
        return carry

    lax.fori_loop(0, STRIPE, zrow, 0)

    def zrow1(i, carry):
        zd_v[pl.ds(i * L, L)] = jnp.zeros((L,), jnp.float32)
        return carry

    lax.fori_loop(0, STRIPE // L, zrow1, 0)
    pltpu.sync_copy(zb_v, num_sh.at[pl.ds(s * STRIPE, STRIPE)])
    pltpu.sync_copy(zd_v, den_sh.at[pl.ds(s * STRIPE, STRIPE)])
    cp_si.wait()
    cp_di.wait()
    plsc.subcore_barrier()

    cp_a = pltpu.async_copy(asrc_hbm.at[si_v], av.at[pl.ds(0, BPW)], sem_a)
    cp_b = pltpu.async_copy(adst_hbm.at[di_v], bv.at[pl.ds(0, BPW)], sem_b)
    cp_r = pltpu.async_copy(xp_hbm.at[si_v], rows_v.at[pl.ds(0, BPW)], sem_r)
    cp_a.wait()
    cp_b.wait()

    def eechunk(i, carry):
        sl = pl.ds(i * L, L)
        z = av[sl] + bv[sl]
        z = jnp.where(z > 0, z, 0.2 * z)
        av[sl] = jnp.exp(z)
        return carry

    lax.fori_loop(0, BPW_UP // L, eechunk, 0)
    pltpu.sync_copy(av.at[pl.ds(0, BPW)], den_sh.at[di_v], add=True)

    cp_r.wait()

    def scalechunk(i, carry):
        svec = av[pl.ds(i * L, L)]
        for r in range(L):
            j = i * L + r
            rows_v[j, :] = rows_v[j, :] * svec[r]
        return carry

    lax.fori_loop(0, BPW_UP // L, scalechunk, 0)
    pltpu.sync_copy(rows_v.at[pl.ds(0, BPW)], num_sh.at[di_v], add=True)
    plsc.subcore_barrier()
    pltpu.sync_copy(
        num_sh.at[pl.ds(s * STRIPE, STRIPE)],
        num_hbm.at[c, pl.ds(s * STRIPE, STRIPE)],
    )
    pltpu.sync_copy(
        den_sh.at[pl.ds(s * STRIPE, STRIPE)],
        den_hbm.at[c, pl.ds(s * STRIPE, STRIPE)],
    )

  return _sc_gat_edges


def _tc_edge_mlp_body(ea_ref, xs_ref, w1_ref, b1_ref, w2_ref, b2_ref, w3_ref,
                      b3_ref, msg_ref):
    f32 = jnp.float32
    ea = ea_ref[...]
    h = jnp.maximum(jnp.dot(ea, w1_ref[...], preferred_element_type=f32)
                    + b1_ref[...], 0.0)
    h = jnp.maximum(jnp.dot(h, w2_ref[...], preferred_element_type=f32)
                    + b2_ref[...], 0.0)
    w = jnp.dot(h, w3_ref[...], preferred_element_type=f32) + b3_ref[...]
    ri = lax.broadcasted_iota(jnp.int32, (F, F * H), 0)
    rj = lax.broadcasted_iota(jnp.int32, (F, F * H), 1)
    R = (rj // H == ri).astype(f32)
    sj = lax.broadcasted_iota(jnp.int32, (F * H, H), 0)
    so = lax.broadcasted_iota(jnp.int32, (F * H, H), 1)
    S = (sj % H == so).astype(f32)
    x2 = jnp.dot(xs_ref[...], R, preferred_element_type=f32)
    msg_ref[...] = jnp.dot(x2 * w, S, preferred_element_type=f32)


_tc_edge_mlp = pl.pallas_call(
    _tc_edge_mlp_body,
    grid=(GE,),
    in_specs=[
        pl.BlockSpec((BE, F), lambda i: (i, 0)),
        pl.BlockSpec((BE, F), lambda i: (i, 0)),
        pl.BlockSpec((F, 128), lambda i: (0, 0)),
        pl.BlockSpec((1, 128), lambda i: (0, 0)),
        pl.BlockSpec((128, 64), lambda i: (0, 0)),
        pl.BlockSpec((1, 64), lambda i: (0, 0)),
        pl.BlockSpec((64, F * H), lambda i: (0, 0)),
        pl.BlockSpec((1, F * H), lambda i: (0, 0)),
    ],
    out_specs=pl.BlockSpec((BE, F), lambda i: (i, 0)),
    out_shape=jax.ShapeDtypeStruct((E, F), jnp.float32),
)


def _tc_node1_body(agg0_ref, agg1_ref, x_ref, root_ref, bias1_ref, wg_ref,
                   ats_ref, atd_ref, xp_ref, as_ref, ad_ref):
    f32 = jnp.float32
    agg = agg0_ref[...] + agg1_ref[...]
    out1 = jnp.maximum(
        agg + jnp.dot(x_ref[...], root_ref[...], preferred_element_type=f32)
        + bias1_ref[...], 0.0)
    xp = jnp.dot(out1, wg_ref[...], preferred_element_type=f32)
    xp_ref[...] = xp
    as_ref[...] = jnp.dot(xp, ats_ref[...], preferred_element_type=f32)
    ad_ref[...] = jnp.dot(xp, atd_ref[...], preferred_element_type=f32)


_tc_node1 = pl.pallas_call(
    _tc_node1_body,
    grid=(GN,),
    in_specs=[
        pl.BlockSpec((BN, F), lambda i: (i, 0)),
        pl.BlockSpec((BN, F), lambda i: (i, 0)),
        pl.BlockSpec((BN, F), lambda i: (i, 0)),
        pl.BlockSpec((F, H), lambda i: (0, 0)),
        pl.BlockSpec((1, H), lambda i: (0, 0)),
        pl.BlockSpec((H, H), lambda i: (0, 0)),
        pl.BlockSpec((H, 1), lambda i: (0, 0)),
        pl.BlockSpec((H, 1), lambda i: (0, 0)),
    ],
    out_specs=[
        pl.BlockSpec((BN, H), lambda i: (i, 0)),
        pl.BlockSpec((BN, 1), lambda i: (i, 0)),
        pl.BlockSpec((BN, 1), lambda i: (i, 0)),
    ],
    out_shape=[
        jax.ShapeDtypeStruct((N_PAD, H), jnp.float32),
        jax.ShapeDtypeStruct((N_PAD, 1), jnp.float32),
        jax.ShapeDtypeStruct((N_PAD, 1), jnp.float32),
    ],
)


def _tc_node2_body(num0_ref, num1_ref, den0_ref, den1_ref, xp_ref, as_ref,
                   ad_ref, batch_ref, biasg_ref, fc1w_ref, fc1b_ref, fc2w_ref,
                   fc2b_ref, out_ref, psum, cnt):
    f32 = jnp.float32
    i = pl.program_id(0)

    @pl.when(i == 0)
    def _init():
        psum[...] = jnp.zeros((G, H), f32)
        cnt[...] = jnp.zeros((G, 1), f32)

    xp = xp_ref[...]
    z = as_ref[...] + ad_ref[...]
    z = jnp.where(z > 0, z, 0.2 * z)
    ee = jnp.exp(z)
    num = num0_ref[...] + num1_ref[...] + ee * xp
    den = den0_ref[...] + den1_ref[...] + ee
    out2 = jnp.maximum(num / den + biasg_ref[...], 0.0)
    b = batch_ref[...]
    oh = (lax.broadcasted_iota(jnp.int32, (G, BN), 0)
          == jnp.broadcast_to(b, (G, BN))).astype(f32)
    psum[...] = psum[...] + jnp.dot(oh, out2, preferred_element_type=f32)
    cnt[...] = cnt[...] + jnp.sum(oh, axis=1, keepdims=True)

    @pl.when(i == GN - 1)
    def _final():
        pooled = psum[...] / jnp.maximum(cnt[...], 1.0)
        hz = jnp.maximum(
            jnp.dot(pooled, fc1w_ref[...], preferred_element_type=f32)
            + fc1b_ref[...], 0.0)
        out_ref[...] = (jnp.dot(hz, fc2w_ref[...], preferred_element_type=f32)
                        + fc2b_ref[...])


_tc_node2 = pl.pallas_call(
    _tc_node2_body,
    grid=(GN,),
    in_specs=[
        pl.BlockSpec((BN, H), lambda i: (i, 0)),
        pl.BlockSpec((BN, H), lambda i: (i, 0)),
        pl.BlockSpec((BN, 1), lambda i: (i, 0)),
        pl.BlockSpec((BN, 1), lambda i: (i, 0)),
        pl.BlockSpec((BN, H), lambda i: (i, 0)),
        pl.BlockSpec((BN, 1), lambda i: (i, 0)),
        pl.BlockSpec((BN, 1), lambda i: (i, 0)),
        pl.BlockSpec((1, BN), lambda i: (0, i)),
        pl.BlockSpec((1, H), lambda i: (0, 0)),
        pl.BlockSpec((H, O), lambda i: (0, 0)),
        pl.BlockSpec((1, O), lambda i: (0, 0)),
        pl.BlockSpec((O, 1), lambda i: (0, 0)),
        pl.BlockSpec((1, 1), lambda i: (0, 0)),
    ],
    out_specs=pl.BlockSpec((G, 1), lambda i: (0, 0)),
    out_shape=jax.ShapeDtypeStruct((G, 1), jnp.float32),
    scratch_shapes=[
        pltpu.VMEM((G, H), jnp.float32),
        pltpu.VMEM((G, 1), jnp.float32),
    ],
)


def kernel(x, edge_index, edge_attr, batch, W1, b1, W2, b2, W3, b3, root,
           bias1, Wg, att_src, att_dst, bias_g, fc1_w, fc1_b, fc2_w, fc2_b):
    src = edge_index[0].astype(jnp.int32)
    dst = edge_index[1].astype(jnp.int32)
    x_p = jnp.pad(x, ((0, N_PAD - N), (0, 0)))
    batch2d = jnp.pad(batch.astype(jnp.int32), (0, N_PAD - N),
                      constant_values=G).reshape(1, N_PAD)

    xs = _build_sc_gather_rows()(x, src)
    msg = _tc_edge_mlp(edge_attr, xs, W1, b1.reshape(1, 128), W2,
                       b2.reshape(1, 64), W3, b3.reshape(1, F * H))
    aggp = _build_sc_scatter_add_rows()(msg, dst)
    xp, asr, adr = _tc_node1(aggp[0], aggp[1], x_p, root, bias1.reshape(1, H),
                             Wg, att_src.reshape(H, 1), att_dst.reshape(H, 1))
    nump, denp = _build_sc_gat_edges()(asr.reshape(N_PAD), adr.reshape(N_PAD),
                                       xp, src, dst)
    out2d = _tc_node2(nump[0], nump[1], denp[0].reshape(N_PAD, 1),
                      denp[1].reshape(N_PAD, 1), xp, asr, adr, batch2d,
                      bias_g.reshape(1, H), fc1_w, fc1_b.reshape(1, O),
                      fc2_w, fc2_b.reshape(1, 1))
    return out2d.reshape(G)


# speedup vs baseline: 15.0808x; 1.1181x over previous
import functools

import jax
import jax.numpy as jnp
from jax import lax
from jax.experimental import pallas as pl
from jax.experimental.pallas import tpu as pltpu
from jax.experimental.pallas import tpu_sc as plsc

N = 10000
E = 160000
F = 16
H = 16
G = 64
O = 64

NC = 2
NS = 16
L = 16
NW = NC * NS

BPW = E // NW
BPW_UP = 5008
N_PAD = 10240
STRIPE = N_PAD // NS

RP = E // 8
RBE = 400
GE = RP // RBE
BN = 1280
GN = N_PAD // BN

_SC_PARAMS = pltpu.CompilerParams(use_tc_tiling_on_sc=False)


def _mesh():
    return plsc.VectorSubcoreMesh(core_axis_name="c", subcore_axis_name="s",
                                  num_cores=NC, num_subcores=NS)


@functools.cache
def _build_sc_gather_rows():
  kern = functools.partial(
    pl.kernel,
    out_type=jax.ShapeDtypeStruct((E, F), jnp.float32),
    mesh=_mesh(),
    compiler_params=_SC_PARAMS,
    scratch_types=[
        pltpu.VMEM((BPW,), jnp.int32),
        pltpu.VMEM((BPW, F), jnp.float32),
        pltpu.SemaphoreType.DMA,
    ],
  )
  @kern
  def _sc_gather_rows(x_hbm, src_hbm, out_hbm, idx_v, rows_v, sem):
    wid = lax.axis_index("s") * NC + lax.axis_index("c")
    base = wid * BPW
    pltpu.sync_copy(src_hbm.at[pl.ds(base, BPW)], idx_v)
    pltpu.async_copy(x_hbm.at[idx_v], rows_v, sem).wait()
    pltpu.sync_copy(rows_v, out_hbm.at[pl.ds(base, BPW)])

  return _sc_gather_rows


@functools.cache
def _build_sc_scatter_add_rows():
  RPW = RP // NW
  RCH = 125
  kern = functools.partial(
    pl.kernel,
    out_type=jax.ShapeDtypeStruct((NC, N_PAD, F), jnp.float32),
    mesh=_mesh(),
    compiler_params=_SC_PARAMS,
    scratch_types=[
        pltpu.VMEM((BPW,), jnp.int32),
        pltpu.VMEM((BPW, F), jnp.float32),
        pltpu.VMEM((RCH, 128), jnp.float32),
        pltpu.VMEM((STRIPE, F), jnp.float32),
        pltpu.VMEM_SHARED((N_PAD, F), jnp.float32),
        pltpu.SemaphoreType.DMA,
        pltpu.SemaphoreType.DMA,
    ],
  )
  @kern
  def _sc_scatter_add_rows(msg_hbm, dst_hbm, out_hbm, idx_v, rows_v, rp_v,
                           zb_v, acc_sh, sem, sem2):
    c = lax.axis_index("c")
    s = lax.axis_index("s")
    wid = s * NC + c
    base = wid * BPW
    rbase = wid * RPW

    cp_idx = pltpu.async_copy(dst_hbm.at[pl.ds(base, BPW)], idx_v, sem)

    def zrow(i, carry):
        zb_v[i, :] = jnp.zeros((L,), jnp.float32)
        return carry

    lax.fori_loop(0, STRIPE, zrow, 0)
    pltpu.sync_copy(zb_v, acc_sh.at[pl.ds(s * STRIPE, STRIPE)])

    for ch in range(RPW // RCH):
        pltpu.sync_copy(msg_hbm.at[pl.ds(rbase + ch * RCH, RCH)], rp_v)

        def unpack(r, carry):
            for k in range(8):
                rows_v[ch * RCH * 8 + r * 8 + k, :] = rp_v[r, pl.ds(k * L, L)]
            return carry

        lax.fori_loop(0, RCH, unpack, 0)

    cp_idx.wait()
    plsc.subcore_barrier()
    pltpu.sync_copy(rows_v, acc_sh.at[idx_v], add=True)
    plsc.subcore_barrier()
    pltpu.sync_copy(
        acc_sh.at[pl.ds(s * STRIPE, STRIPE)],
        out_hbm.at[c, pl.ds(s * STRIPE, STRIPE)],
    )

  return _sc_scatter_add_rows


@functools.cache
def _build_sc_gat_edges():
  kern = functools.partial(
    pl.kernel,
    out_type=[
        jax.ShapeDtypeStruct((NC, N_PAD, F), jnp.float32),
        jax.ShapeDtypeStruct((NC, N_PAD), jnp.float32),
    ],
    mesh=_mesh(),
    compiler_params=_SC_PARAMS,
    scratch_types=[
        pltpu.VMEM((BPW,), jnp.int32),
        pltpu.VMEM((BPW,), jnp.int32),
        pltpu.VMEM((BPW_UP,), jnp.float32),
        pltpu.VMEM((BPW_UP,), jnp.float32),
        pltpu.VMEM((BPW_UP, F), jnp.float32),
        pltpu.VMEM((STRIPE, F), jnp.float32),
        pltpu.VMEM((STRIPE,), jnp.float32),
        pltpu.VMEM_SHARED((N_PAD, F), jnp.float32),
        pltpu.VMEM_SHARED((N_PAD,), jnp.float32),
        pltpu.SemaphoreType.DMA,
        pltpu.SemaphoreType.DMA,
        pltpu.SemaphoreType.DMA,
    ],
  )
  @kern
  def _sc_gat_edges(asrc_hbm, adst_hbm, xp_hbm, src_hbm, dst_hbm,
                    num_hbm, den_hbm,
                    si_v, di_v, av, bv, rows_v, zb_v, zd_v, num_sh, den_sh,
                    sem_a, sem_b, sem_r):
    c = lax.axis_index("c")
    s = lax.axis_index("s")
    wid = s * NC + c
    base = wid * BPW

    cp_si = pltpu.async_copy(src_hbm.at[pl.ds(base, BPW)], si_v, sem_a)
    cp_di = pltpu.async_copy(dst_hbm.at[pl.ds(base, BPW)], di_v, sem_b)

    def zrow(i, carry):
        zb_v[i, :] = jnp.zeros((L,), jnp.float32)
        return carry

    lax.fori_loop(0, STRIPE, zrow, 0)

    def zrow1(i, carry):
        zd_v[pl.ds(i * L, L)] = jnp.zeros((L,), jnp.float32)
        return carry

    lax.fori_loop(0, STRIPE // L, zrow1, 0)
    pltpu.sync_copy(zb_v, num_sh.at[pl.ds(s * STRIPE, STRIPE)])
    pltpu.sync_copy(zd_v, den_sh.at[pl.ds(s * STRIPE, STRIPE)])
    cp_si.wait()
    cp_di.wait()
    plsc.subcore_barrier()

    cp_a = pltpu.async_copy(asrc_hbm.at[si_v], av.at[pl.ds(0, BPW)], sem_a)
    cp_b = pltpu.async_copy(adst_hbm.at[di_v], bv.at[pl.ds(0, BPW)], sem_b)
    cp_r = pltpu.async_copy(xp_hbm.at[si_v], rows_v.at[pl.ds(0, BPW)], sem_r)
    cp_a.wait()
    cp_b.wait()

    def eechunk(i, carry):
        sl = pl.ds(i * L, L)
        z = av[sl] + bv[sl]
        z = jnp.where(z > 0, z, 0.2 * z)
        av[sl] = jnp.exp(z)
        return carry

    lax.fori_loop(0, BPW_UP // L, eechunk, 0)
    pltpu.sync_copy(av.at[pl.ds(0, BPW)], den_sh.at[di_v], add=True)

    cp_r.wait()

    def scalechunk(i, carry):
        svec = av[pl.ds(i * L, L)]
        for r in range(L):
            j = i * L + r
            rows_v[j, :] = rows_v[j, :] * svec[r]
        return carry

    lax.fori_loop(0, BPW_UP // L, scalechunk, 0)
    pltpu.sync_copy(rows_v.at[pl.ds(0, BPW)], num_sh.at[di_v], add=True)
    plsc.subcore_barrier()
    pltpu.sync_copy(
        num_sh.at[pl.ds(s * STRIPE, STRIPE)],
        num_hbm.at[c, pl.ds(s * STRIPE, STRIPE)],
    )
    pltpu.sync_copy(
        den_sh.at[pl.ds(s * STRIPE, STRIPE)],
        den_hbm.at[c, pl.ds(s * STRIPE, STRIPE)],
    )

  return _sc_gat_edges


def _tc_edge_mlp_body(ea_ref, xs_ref, w1_ref, b1_ref, w2_ref, b2_ref, w3_ref,
                      b3_ref, r_ref, s_ref, msg_ref):
    f32 = jnp.float32
    h = jnp.maximum(jnp.dot(ea_ref[...], w1_ref[...], preferred_element_type=f32)
                    + b1_ref[...], 0.0)
    h = jnp.maximum(jnp.dot(h, w2_ref[...], preferred_element_type=f32)
                    + b2_ref[...], 0.0)
    w = jnp.dot(h, w3_ref[...], preferred_element_type=f32) + b3_ref[...]
    x2 = jnp.dot(xs_ref[...], r_ref[...], preferred_element_type=f32)
    msg_ref[...] = jnp.dot(x2 * w, s_ref[...], preferred_element_type=f32)


_tc_edge_mlp = pl.pallas_call(
    _tc_edge_mlp_body,
    grid=(GE,),
    in_specs=[
        pl.BlockSpec((RBE, 128), lambda i: (i, 0)),
        pl.BlockSpec((RBE, 128), lambda i: (i, 0)),
        pl.BlockSpec((128, 1024), lambda i: (0, 0)),
        pl.BlockSpec((1, 1024), lambda i: (0, 0)),
        pl.BlockSpec((1024, 512), lambda i: (0, 0)),
        pl.BlockSpec((1, 512), lambda i: (0, 0)),
        pl.BlockSpec((512, 2048), lambda i: (0, 0)),
        pl.BlockSpec((1, 2048), lambda i: (0, 0)),
        pl.BlockSpec((128, 2048), lambda i: (0, 0)),
        pl.BlockSpec((2048, 128), lambda i: (0, 0)),
    ],
    out_specs=pl.BlockSpec((RBE, 128), lambda i: (i, 0)),
    out_shape=jax.ShapeDtypeStruct((RP, 128), jnp.float32),
)


def _tc_node1_body(agg0_ref, agg1_ref, x_ref, root_ref, bias1_ref, wg_ref,
                   ats_ref, atd_ref, xp_ref, as_ref, ad_ref):
    f32 = jnp.float32
    agg = agg0_ref[...] + agg1_ref[...]
    out1 = jnp.maximum(
        agg + jnp.dot(x_ref[...], root_ref[...], preferred_element_type=f32)
        + bias1_ref[...], 0.0)
    xp = jnp.dot(out1, wg_ref[...], preferred_element_type=f32)
    xp_ref[...] = xp
    as_ref[...] = jnp.dot(xp, ats_ref[...], preferred_element_type=f32)
    ad_ref[...] = jnp.dot(xp, atd_ref[...], preferred_element_type=f32)


_tc_node1 = pl.pallas_call(
    _tc_node1_body,
    grid=(GN,),
    in_specs=[
        pl.BlockSpec((BN, F), lambda i: (i, 0)),
        pl.BlockSpec((BN, F), lambda i: (i, 0)),
        pl.BlockSpec((BN, F), lambda i: (i, 0)),
        pl.BlockSpec((F, H), lambda i: (0, 0)),
        pl.BlockSpec((1, H), lambda i: (0, 0)),
        pl.BlockSpec((H, H), lambda i: (0, 0)),
        pl.BlockSpec((H, 1), lambda i: (0, 0)),
        pl.BlockSpec((H, 1), lambda i: (0, 0)),
    ],
    out_specs=[
        pl.BlockSpec((BN, H), lambda i: (i, 0)),
        pl.BlockSpec((BN, 1), lambda i: (i, 0)),
        pl.BlockSpec((BN, 1), lambda i: (i, 0)),
    ],
    out_shape=[
        jax.ShapeDtypeStruct((N_PAD, H), jnp.float32),
        jax.ShapeDtypeStruct((N_PAD, 1), jnp.float32),
        jax.ShapeDtypeStruct((N_PAD, 1), jnp.float32),
    ],
)


def _tc_node2_body(num0_ref, num1_ref, den0_ref, den1_ref, xp_ref, as_ref,
                   ad_ref, batch_ref, biasg_ref, fc1w_ref, fc1b_ref, fc2w_ref,
                   fc2b_ref, out_ref, psum, cnt):
    f32 = jnp.float32
    i = pl.program_id(0)

    @pl.when(i == 0)
    def _init():
        psum[...] = jnp.zeros((G, H), f32)
        cnt[...] = jnp.zeros((G, 1), f32)

    xp = xp_ref[...]
    z = as_ref[...] + ad_ref[...]
    z = jnp.where(z > 0, z, 0.2 * z)
    ee = jnp.exp(z)
    num = num0_ref[...] + num1_ref[...] + ee * xp
    den = den0_ref[...] + den1_ref[...] + ee
    out2 = jnp.maximum(num / den + biasg_ref[...], 0.0)
    b = batch_ref[...]
    oh = (lax.broadcasted_iota(jnp.int32, (G, BN), 0)
          == jnp.broadcast_to(b, (G, BN))).astype(f32)
    psum[...] = psum[...] + jnp.dot(oh, out2, preferred_element_type=f32)
    cnt[...] = cnt[...] + jnp.sum(oh, axis=1, keepdims=True)

    @pl.when(i == GN - 1)
    def _final():
        pooled = psum[...] / jnp.maximum(cnt[...], 1.0)
        hz = jnp.maximum(
            jnp.dot(pooled, fc1w_ref[...], preferred_element_type=f32)
            + fc1b_ref[...], 0.0)
        out_ref[...] = (jnp.dot(hz, fc2w_ref[...], preferred_element_type=f32)
                        + fc2b_ref[...])


_tc_node2 = pl.pallas_call(
    _tc_node2_body,
    grid=(GN,),
    in_specs=[
        pl.BlockSpec((BN, H), lambda i: (i, 0)),
        pl.BlockSpec((BN, H), lambda i: (i, 0)),
        pl.BlockSpec((BN, 1), lambda i: (i, 0)),
        pl.BlockSpec((BN, 1), lambda i: (i, 0)),
        pl.BlockSpec((BN, H), lambda i: (i, 0)),
        pl.BlockSpec((BN, 1), lambda i: (i, 0)),
        pl.BlockSpec((BN, 1), lambda i: (i, 0)),
        pl.BlockSpec((1, BN), lambda i: (0, i)),
        pl.BlockSpec((1, H), lambda i: (0, 0)),
        pl.BlockSpec((H, O), lambda i: (0, 0)),
        pl.BlockSpec((1, O), lambda i: (0, 0)),
        pl.BlockSpec((O, 1), lambda i: (0, 0)),
        pl.BlockSpec((1, 1), lambda i: (0, 0)),
    ],
    out_specs=pl.BlockSpec((G, 1), lambda i: (0, 0)),
    out_shape=jax.ShapeDtypeStruct((G, 1), jnp.float32),
    scratch_shapes=[
        pltpu.VMEM((G, H), jnp.float32),
        pltpu.VMEM((G, 1), jnp.float32),
    ],
)


def kernel(x, edge_index, edge_attr, batch, W1, b1, W2, b2, W3, b3, root,
           bias1, Wg, att_src, att_dst, bias_g, fc1_w, fc1_b, fc2_w, fc2_b):
    src = edge_index[0].astype(jnp.int32)
    dst = edge_index[1].astype(jnp.int32)
    x_p = jnp.pad(x, ((0, N_PAD - N), (0, 0)))
    batch2d = jnp.pad(batch.astype(jnp.int32), (0, N_PAD - N),
                      constant_values=G).reshape(1, N_PAD)

    eye8 = jnp.eye(8, dtype=jnp.float32)
    ri = lax.broadcasted_iota(jnp.int32, (F, F * H), 0)
    rj = lax.broadcasted_iota(jnp.int32, (F, F * H), 1)
    R16 = (rj // H == ri).astype(jnp.float32)
    sj = lax.broadcasted_iota(jnp.int32, (F * H, H), 0)
    so = lax.broadcasted_iota(jnp.int32, (F * H, H), 1)
    S16 = (sj % H == so).astype(jnp.float32)
    ea_p8 = edge_attr.reshape(RP, 128)

    xs = _build_sc_gather_rows()(x, src)
    msgp = _tc_edge_mlp(ea_p8, xs.reshape(RP, 128),
                        jnp.kron(eye8, W1), jnp.tile(b1, 8).reshape(1, 1024),
                        jnp.kron(eye8, W2), jnp.tile(b2, 8).reshape(1, 512),
                        jnp.kron(eye8, W3), jnp.tile(b3, 8).reshape(1, 2048),
                        jnp.kron(eye8, R16), jnp.kron(eye8, S16))
    aggp = _build_sc_scatter_add_rows()(msgp, dst)
    xp, asr, adr = _tc_node1(aggp[0], aggp[1], x_p, root, bias1.reshape(1, H),
                             Wg, att_src.reshape(H, 1), att_dst.reshape(H, 1))
    nump, denp = _build_sc_gat_edges()(asr.reshape(N_PAD), adr.reshape(N_PAD),
                                       xp, src, dst)
    out2d = _tc_node2(nump[0], nump[1], denp[0].reshape(N_PAD, 1),
                      denp[1].reshape(N_PAD, 1), xp, asr, adr, batch2d,
                      bias_g.reshape(1, H), fc1_w, fc1_b.reshape(1, O),
                      fc2_w, fc2_b.reshape(1, 1))
    return out2d.reshape(G)
